# baseline (speedup 1.0000x reference)
import jax
import jax.numpy as jnp
from jax import lax
from jax.experimental import pallas as pl
from jax.experimental.pallas import tpu as pltpu
from jax.experimental.pallas import tpu_sc as plsc

K = 129
D_MODEL = 1024
NUM_HEADS = 16
SCALE = 32.0

HALF0 = 64
HALF1 = K - HALF0
RPS = 8


def _scale_rows(buf, nrows):
    for r in range(nrows):
        def body(i, carry, r=r):
            sl = pl.ds(i * 16, 16)
            buf[r, sl] = buf[r, sl] * SCALE
            return carry
        lax.fori_loop(0, D_MODEL // 16, body, 0)


def _body(wq, wk, wv, oq, ok, ov, sh0, sh1, sh2, buf, sem):
    s = lax.axis_index("s")
    c = lax.axis_index("c")
    shared = (sh0, sh1, sh2)
    tabs = (wq, wk, wv)
    outs = (oq, ok, ov)

    @pl.when(s < 8)
    def _stage():
        base = c * HALF0 + s * RPS
        for t in range(3):
            pltpu.sync_copy(tabs[t].at[pl.ds(base, RPS)], buf)
            _scale_rows(buf, RPS)
            pltpu.sync_copy(buf, shared[t].at[pl.ds(s * RPS, RPS)])

    plsc.subcore_barrier()

    @pl.when(c == 0)
    def _lo():
        descs = [
            pltpu.async_copy(shared[t].at[pl.ds(0, HALF0)],
                             outs[t].at[0, s, pl.ds(0, HALF0)], sem)
            for t in range(3)
        ]
        for d in descs:
            d.wait()

    @pl.when(c == 1)
    def _hi():
        descs = [
            pltpu.async_copy(shared[t].at[pl.ds(0, HALF0)],
                             outs[t].at[0, s, pl.ds(HALF0, HALF0)], sem)
            for t in range(3)
        ]
        for t in range(3):
            pltpu.sync_copy(tabs[t].at[pl.ds(K - 1, 1)], buf.at[pl.ds(0, 1)])
            _scale_rows(buf, 1)
            pltpu.sync_copy(buf.at[pl.ds(0, 1)],
                            outs[t].at[0, s, pl.ds(K - 1, 1)])
        for d in descs:
            d.wait()


def kernel(Wq, Wk, Wv):
    out = jax.ShapeDtypeStruct((1, NUM_HEADS, K, D_MODEL), jnp.float32)
    mesh = plsc.VectorSubcoreMesh(core_axis_name="c", subcore_axis_name="s")
    f = pl.kernel(
        _body,
        out_type=[out, out, out],
        mesh=mesh,
        scratch_types=[
            pltpu.VMEM_SHARED((HALF0, D_MODEL), jnp.float32),
            pltpu.VMEM_SHARED((HALF0, D_MODEL), jnp.float32),
            pltpu.VMEM_SHARED((HALF0, D_MODEL), jnp.float32),
            pltpu.VMEM((RPS, D_MODEL), jnp.float32),
            pltpu.SemaphoreType.DMA,
        ],
    )
    return tuple(f(Wq, Wk, Wv))
